# baseline (speedup 1.0000x reference)
import functools

import jax
import jax.numpy as jnp
from jax import lax
from jax.experimental import pallas as pl
from jax.experimental.pallas import tpu as pltpu
from jax.experimental.pallas import tpu_sc as plsc

B = 16384
D = 16
H = 32
NGRP = 125000

NC = 2
NS = 16
NW = NC * NS
BPW = B // NW
NCHUNK = 4
CHUNK = BPW // NCHUNK

_W1_OFF = 64
_B1_OFF = _W1_OFF + H * D
_W2_OFF = _B1_OFF + H
_B2_OFF = _W2_OFF + H
_WLEN = 704

_MESH = plsc.VectorSubcoreMesh(core_axis_name="c", subcore_axis_name="s")


@functools.partial(
    pl.kernel,
    out_type=jax.ShapeDtypeStruct((B,), jnp.float32),
    mesh=_MESH,
    scratch_types=[
        pltpu.VMEM((BPW,), jnp.int32),
        pltpu.VMEM((NCHUNK, CHUNK), jnp.int32),
        pltpu.VMEM((NCHUNK, CHUNK, 128), jnp.float32),
        pltpu.VMEM((_WLEN,), jnp.float32),
        pltpu.VMEM((BPW,), jnp.float32),
        pltpu.SemaphoreType.DMA,
    ],
    compiler_params=pltpu.CompilerParams(needs_layout_passes=False),
)
def _sc_fused(x_hbm, g_hbm, table_hbm, w_hbm, out_hbm, idx_v, g_v, rows_v, w_v, out_v, sem):
    c = lax.axis_index("c")
    s = lax.axis_index("s")
    wid = s * NC + c
    base = wid * BPW

    pltpu.sync_copy(x_hbm.at[pl.ds(base, BPW)], idx_v)
    pltpu.sync_copy(g_hbm.at[wid], g_v)
    pltpu.sync_copy(w_hbm, w_v)

    iota = lax.iota(jnp.int32, 16)

    copies = [
        pltpu.make_async_copy(table_hbm.at[g_v.at[k]], rows_v.at[k], sem)
        for k in range(NCHUNK)
    ]
    for cp in copies:
        cp.start()
    for cp in copies:
        cp.wait()

    def _wsplat(off):
        return plsc.load_gather(w_v, [jnp.full((16,), off, jnp.int32)])

    def grp_body(g, _):
        k = g >> 3
        t = g & 7
        kvec = jnp.full((16,), k, jnp.int32)
        rowi = iota + t * 16
        idx16 = idx_v[pl.ds(g * 16, 16)]
        colb = (idx16 & 7) * 16
        cols = [
            plsc.load_gather(rows_v, [kvec, rowi, colb + d]) for d in range(D)
        ]
        acc = _wsplat(_B2_OFF)
        for j in range(H):
            h = _wsplat(_B1_OFF + j)
            for d in range(D):
                h = h + _wsplat(_W1_OFF + j * D + d) * cols[d]
            h = jnp.maximum(h, 0.0)
            acc = acc + _wsplat(_W2_OFF + j) * h
        out_v[pl.ds(g * 16, 16)] = acc
        return 0

    lax.fori_loop(0, BPW // 16, grp_body, 0)
    pltpu.sync_copy(out_v, out_hbm.at[pl.ds(base, BPW)])


def kernel(x, table, W1, b1, W2, b2):
    idx = x.astype(jnp.int32)
    g = (idx >> 3).reshape(NW, NCHUNK, CHUNK)
    table2 = table.reshape(NGRP, 128)
    wflat = jnp.concatenate([
        jnp.zeros((_W1_OFF,), jnp.float32),
        W1.reshape(-1), b1, W2.reshape(-1), b2,
        jnp.zeros((_WLEN - _B2_OFF - 1,), jnp.float32),
    ])
    out = _sc_fused(idx, g, table2, wflat)
    return out.reshape(B, 1)
